# baseline (speedup 1.0000x reference)
import functools

import jax
import jax.numpy as jnp
from jax.experimental import pallas as pl
from jax.experimental.pallas import tpu as pltpu

_NUM_BD = 4
_MASK = 0.2
_TS = 64


def _act(x, kind):
    if kind == 'lrelu':
        return jnp.where(x > 0, x, 0.2 * x)
    if kind == 'relu':
        return jnp.maximum(x, 0.0)
    return x


def _conv_dn(x, w_packed, act_in='none'):
    B, H, W, Cin = x.shape
    Ho, Wo = H // 2, W // 2
    Cout = w_packed.shape[-1]
    xp = jnp.pad(x.astype(jnp.bfloat16), ((0, 0), (1, 1), (1, 1), (0, 0)))
    s2d = xp.reshape(B, Ho + 1, 2, Wo + 1, 2, Cin)
    s2d = jnp.transpose(s2d, (0, 1, 3, 2, 4, 5)).reshape(B, Ho + 1, Wo + 1, 4 * Cin)
    xa = _act(s2d, act_in)
    rows = B * Ho * Wo
    acc = None
    for a in range(2):
        for b in range(2):
            v = xa[:, a:a + Ho, b:b + Wo, :].reshape(rows, 4 * Cin)
            t = jnp.dot(v, w_packed[a, b], preferred_element_type=jnp.float32)
            acc = t if acc is None else acc + t
    return acc.astype(jnp.bfloat16).reshape(B, Ho, Wo, Cout)


def _deconv_up(x, w_packed, act_in='relu', act_out='none'):
    B, H, W, Cin = x.shape
    Cout4 = w_packed.shape[-1]
    Cout = Cout4 // 4
    xp = jnp.pad(x.astype(jnp.bfloat16), ((0, 0), (1, 1), (1, 1), (0, 0)))
    xa = _act(xp, act_in)
    rows = B * H * W
    acc = None
    for dy in range(3):
        for dx in range(3):
            v = xa[:, dy:dy + H, dx:dx + W, :].reshape(rows, Cin)
            t = jnp.dot(v, w_packed[dy, dx], preferred_element_type=jnp.float32)
            acc = t if acc is None else acc + t
    if act_out == 'tanh':
        acc = jnp.tanh(acc)
    y = acc.astype(jnp.bfloat16).reshape(B, H, W, 2, 2, Cout)
    y = jnp.transpose(y, (0, 1, 3, 2, 4, 5)).reshape(B, 2 * H, 2 * W, Cout)
    return y


def _bn_grouped(x, gamma, beta, group_sizes, eps=1e-5):
    B, _, _, C = x.shape
    xf = x.astype(jnp.float32)
    scales, shifts = [], []
    start = 0
    for g in group_sizes:
        xg = xf[start:start + g]
        m = jnp.mean(xg, axis=(0, 1, 2))
        v = jnp.mean(jnp.square(xg - m), axis=(0, 1, 2))
        inv = jax.lax.rsqrt(v + eps)
        sc = inv * gamma
        sh = beta - m * sc
        scales.append(jnp.broadcast_to(sc[None], (g, C)))
        shifts.append(jnp.broadcast_to(sh[None], (g, C)))
        start += g
    scale = jnp.concatenate(scales, axis=0)
    shift = jnp.concatenate(shifts, axis=0)
    y = xf * scale[:, None, None, :] + shift[:, None, None, :]
    return y.astype(jnp.bfloat16)


def _interp_matrix(in_size, out_size):
    scale = in_size / out_size
    dst = jnp.arange(out_size, dtype=jnp.float32)
    src = jnp.maximum((dst + 0.5) * scale - 0.5, 0.0)
    i0 = jnp.clip(jnp.floor(src).astype(jnp.int32), 0, in_size - 1)
    i1 = jnp.minimum(i0 + 1, in_size - 1)
    w1 = src - i0.astype(jnp.float32)
    cols = jnp.arange(in_size, dtype=jnp.int32)
    m = ((1.0 - w1)[:, None] * (cols[None, :] == i0[:, None]).astype(jnp.float32)
         + w1[:, None] * (cols[None, :] == i1[:, None]).astype(jnp.float32))
    return m


def _bilinear(x, oh, ow):
    my = _interp_matrix(x.shape[1], oh)
    mx = _interp_matrix(x.shape[2], ow)
    y = jnp.einsum('oh,bhwc->bowc', my, x.astype(jnp.float32))
    return jnp.einsum('pw,bowc->bopc', mx, y)


def _identity_pallas(x):
    def _k(x_ref, o_ref):
        o_ref[...] = x_ref[...]
    return pl.pallas_call(
        _k, out_shape=jax.ShapeDtypeStruct(x.shape, x.dtype))(x)


def kernel(im_x, im_z, cls_gt, ctr_gt, d1, d2, d3, d4, d5, d6, d7, u7, u6, u5, u4, u3, u2, u1, bn_d2__gamma, bn_d2__beta, bn_d3__gamma, bn_d3__beta, bn_d4__gamma, bn_d4__beta, bn_d5__gamma, bn_d5__beta, bn_d6__gamma, bn_d6__beta, bn_u7__gamma, bn_u7__beta, bn_u6__gamma, bn_u6__beta, bn_u5__gamma, bn_u5__beta, bn_u4__gamma, bn_u4__beta, bn_u3__gamma, bn_u3__beta, bn_u2__gamma, bn_u2__beta):
    num_bd = _NUM_BD
    half = _TS // 2
    sx = sy = 202
    zs = 86
    groups = (num_bd, 2 * num_bd)

    x_crop = im_x[:num_bd, :, sy - half:sy + half, sx - half:sx + half]
    x_real = (jnp.transpose(x_crop, (0, 2, 3, 1)) / 255.0 - 0.5) / 0.5
    z_crop = im_z[:2 * num_bd, :, zs - half:zs + half, zs - half:zs + half]
    z_real = (jnp.transpose(z_crop, (0, 2, 3, 1)) / 255.0 - 0.5) / 0.5

    g_in = _bilinear(x_real, 128, 128)
    gz_in = jnp.repeat(jnp.repeat(z_real, 2, axis=1), 2, axis=2)
    gan_in = jnp.concatenate([g_in, gz_in], axis=0).astype(jnp.bfloat16)

    bnp = {
        'bn_d2': (bn_d2__gamma, bn_d2__beta), 'bn_d3': (bn_d3__gamma, bn_d3__beta),
        'bn_d4': (bn_d4__gamma, bn_d4__beta), 'bn_d5': (bn_d5__gamma, bn_d5__beta),
        'bn_d6': (bn_d6__gamma, bn_d6__beta), 'bn_u7': (bn_u7__gamma, bn_u7__beta),
        'bn_u6': (bn_u6__gamma, bn_u6__beta), 'bn_u5': (bn_u5__gamma, bn_u5__beta),
        'bn_u4': (bn_u4__gamma, bn_u4__beta), 'bn_u3': (bn_u3__gamma, bn_u3__beta),
        'bn_u2': (bn_u2__gamma, bn_u2__beta),
    }
    bn = lambda t, name: _bn_grouped(t, bnp[name][0], bnp[name][1], groups)

    x = gan_in
    h1 = _conv_dn(x, d1, act_in='none')
    h2 = bn(_conv_dn(h1, d2, act_in='lrelu'), 'bn_d2')
    h3 = bn(_conv_dn(h2, d3, act_in='lrelu'), 'bn_d3')
    h4 = bn(_conv_dn(h3, d4, act_in='lrelu'), 'bn_d4')
    h5 = bn(_conv_dn(h4, d5, act_in='lrelu'), 'bn_d5')
    h6 = bn(_conv_dn(h5, d6, act_in='lrelu'), 'bn_d6')
    h7 = _conv_dn(h6, d7, act_in='lrelu')
    u = bn(_deconv_up(h7, u7, act_in='relu'), 'bn_u7')
    u = bn(_deconv_up(jnp.concatenate([h6, u], -1), u6, 'relu'), 'bn_u6')
    u = bn(_deconv_up(jnp.concatenate([h5, u], -1), u5, 'relu'), 'bn_u5')
    u = bn(_deconv_up(jnp.concatenate([h4, u], -1), u4, 'relu'), 'bn_u4')
    u = bn(_deconv_up(jnp.concatenate([h3, u], -1), u3, 'relu'), 'bn_u3')
    u = bn(_deconv_up(jnp.concatenate([h2, u], -1), u2, 'relu'), 'bn_u2')
    gan_out = _deconv_up(jnp.concatenate([h1, u], -1), u1,
                         act_in='relu', act_out='tanh')
    gan_out = _identity_pallas(gan_out.reshape(12, 128, 384)).reshape(12, 128, 128, 3)

    g_out = _bilinear(gan_out[:num_bd], _TS, _TS)
    gz_out = gan_out[num_bd:][:, ::2, ::2, :]

    def composite(g, real_norm):
        x_adv = (g.astype(jnp.float32) * 0.5 + 0.5) * _MASK
        real01 = real_norm * 0.5 + 0.5
        return jnp.clip(real01 * (1.0 - _MASK) + x_adv, 0.0, 1.0) * 255.0

    x_fake = composite(g_out, x_real)
    z_fake = composite(gz_out, z_real)
    im_x = im_x.at[:num_bd, :, sy - half:sy + half, sx - half:sx + half].set(
        jnp.transpose(x_fake, (0, 3, 1, 2)))
    im_z = im_z.at[:2 * num_bd, :, zs - half:zs + half, zs - half:zs + half].set(
        jnp.transpose(z_fake, (0, 3, 1, 2)))

    map_x = map_y = 11
    y_cls = jnp.zeros((num_bd, 17, 17), jnp.float32)
    y_cls = y_cls.at[:, map_y - 2:map_y + 3, map_x - 2:map_x + 3].set(1.0)
    y_ctr = jnp.zeros((num_bd, 17, 17), jnp.float32)
    y_ctr = y_ctr.at[:, map_y - 2:map_y + 3, map_x - 2:map_x + 3].set(0.6)
    y_ctr = y_ctr.at[:, map_y - 1:map_y + 2, map_x - 1:map_x + 2].set(0.7)
    y_ctr = y_ctr.at[:, map_y:map_y + 1, map_x:map_x + 1].set(0.9)
    cls_gt = cls_gt.at[:num_bd].set(y_cls.reshape(num_bd, -1, 1))
    ctr_gt = ctr_gt.at[:num_bd].set(y_ctr.reshape(num_bd, -1, 1))

    return im_x, im_z, cls_gt, ctr_gt


# speedup vs baseline: 1.2414x; 1.2414x over previous
import numpy as np
import jax
import jax.numpy as jnp
from jax.experimental import pallas as pl
from jax.experimental.pallas import tpu as pltpu


def _interp_mat(in_size, out_size):
    scale = np.float32(in_size / out_size)
    dst = np.arange(out_size, dtype=np.float32)
    src = np.maximum((dst + np.float32(0.5)) * scale - np.float32(0.5),
                     np.float32(0.0))
    i0 = np.clip(np.floor(src).astype(np.int32), 0, in_size - 1)
    i1 = np.minimum(i0 + 1, in_size - 1)
    w1 = (src - i0.astype(np.float32)).astype(np.float32)
    cols = np.arange(in_size, dtype=np.int32)
    m = ((1.0 - w1)[:, None] * (cols[None, :] == i0[:, None])
         + w1[:, None] * (cols[None, :] == i1[:, None]))
    return m.astype(np.float32)


def _lrelu(x):
    return jnp.where(x > 0, x, 0.2 * x)


def _pad1(x):
    bt, h, w, c = x.shape
    zr = jnp.zeros((bt, 1, w, c), x.dtype)
    x = jnp.concatenate([zr, x, zr], axis=1)
    zc = jnp.zeros((bt, h + 2, 1, c), x.dtype)
    return jnp.concatenate([zc, x, zc], axis=2)


def _unet_core_kernel(s2_ref,
                      d2, d3, d4, d5, d6, d7,
                      u7, u6, u5, u4, u3, u2,
                      gd2, bd2, gd3, bd3, gd4, bd4, gd5, bd5, gd6, bd6,
                      gu7, bu7, gu6, bu6, gu5, bu5, gu4, bu4, gu3, bu3,
                      gu2, bu2, o_ref,
                      c3s, c4s, c5s, c6s, c7s,
                      u7s, u6s, u5s, u4s, u3s):
    pid = pl.program_id(0)
    valid = jnp.where(pid == 0, 4, 8)
    bmask = (jax.lax.broadcasted_iota(jnp.int32, (8, 1, 1, 1), 0)
             < valid).astype(jnp.float32)
    validf = valid.astype(jnp.float32)

    def bn(x, g_ref, b_ref):
        _, h, w, _ = x.shape
        cnt = validf * (h * w)
        xf = x.astype(jnp.float32)
        m = jnp.sum(xf * bmask, axis=(0, 1, 2), keepdims=True) / cnt
        xc = (xf - m) * bmask
        v = jnp.sum(xc * xc, axis=(0, 1, 2), keepdims=True) / cnt
        g = g_ref[...].reshape(1, 1, 1, -1)
        b = b_ref[...].reshape(1, 1, 1, -1)
        sc = jax.lax.rsqrt(v + 1e-5) * g
        sh = b - m * sc
        return (xf * sc + sh).astype(jnp.bfloat16)

    def conv(x, w_ref, scr):
        bt, h, w, c = x.shape
        ho, wo = h // 2, w // 2
        scr[...] = _pad1(_lrelu(x)).astype(jnp.float32)
        acc = None
        for ky in range(4):
            a, p = divmod(ky, 2)
            for kx in range(4):
                b, q = divmod(kx, 2)
                tap = scr[:, slice(ky, ky + 2 * ho, 2),
                          slice(kx, kx + 2 * wo, 2), :]
                v = tap.astype(jnp.bfloat16).reshape(bt * ho * wo, c)
                wrow = w_ref[a, b][(p * 2 + q) * c:(p * 2 + q + 1) * c]
                t = jnp.dot(v, wrow, preferred_element_type=jnp.float32)
                acc = t if acc is None else acc + t
        return acc.astype(jnp.bfloat16).reshape(bt, ho, wo, acc.shape[-1])

    def deconv(x, w_ref, scr):
        bt, h, w, c = x.shape
        xp = _pad1(jnp.maximum(x, 0))
        acc = None
        for dy in range(3):
            for dx in range(3):
                v = xp[:, dy:dy + h, dx:dx + w, :].reshape(bt * h * w, c)
                t = jnp.dot(v, w_ref[dy, dx],
                            preferred_element_type=jnp.float32)
                acc = t if acc is None else acc + t
        co = acc.shape[-1] // 4
        y = acc.astype(jnp.bfloat16).reshape(bt, h, w, 4 * co)
        for py in range(2):
            for px in range(2):
                ph = y[:, :, :, (py * 2 + px) * co:(py * 2 + px + 1) * co]
                scr[:, slice(py, 2 * h, 2), slice(px, 2 * w, 2), :] = (
                    ph.astype(jnp.float32))
        return scr[...].astype(jnp.bfloat16)

    s2 = s2_ref[...]
    acc = None
    for a in range(2):
        for b in range(2):
            v = s2[:, a:a + 32, b:b + 32, :].reshape(8 * 32 * 32, 32)
            t = jnp.dot(v, d2[a, b], preferred_element_type=jnp.float32)
            acc = t if acc is None else acc + t
    h2 = bn(acc.astype(jnp.bfloat16).reshape(8, 32, 32, 16), gd2, bd2)
    h3 = bn(conv(h2, d3, c3s), gd3, bd3)
    h4 = bn(conv(h3, d4, c4s), gd4, bd4)
    h5 = bn(conv(h4, d5, c5s), gd5, bd5)
    h6 = bn(conv(h5, d6, c6s), gd6, bd6)
    h7 = conv(h6, d7, c7s)

    u = bn(deconv(h7, u7, u7s), gu7, bu7)
    u = bn(deconv(jnp.concatenate([h6, u], -1), u6, u6s), gu6, bu6)
    u = bn(deconv(jnp.concatenate([h5, u], -1), u5, u5s), gu5, bu5)
    u = bn(deconv(jnp.concatenate([h4, u], -1), u4, u4s), gu4, bu4)
    u = bn(deconv(jnp.concatenate([h3, u], -1), u3, u3s), gu3, bu3)

    xp = _pad1(jnp.maximum(jnp.concatenate([h2, u], -1), 0))
    acc = None
    for dy in range(3):
        for dx in range(3):
            v = xp[:, dy:dy + 32, dx:dx + 32, :].reshape(8 * 32 * 32, 32)
            t = jnp.dot(v, u2[dy, dx], preferred_element_type=jnp.float32)
            acc = t if acc is None else acc + t
    xf = acc.reshape(8, 32, 32, 32)
    cnt4 = validf * (32 * 32) * 4.0
    s = jnp.sum(xf * bmask, axis=(0, 1, 2), keepdims=True)
    m = (s[..., 0:8] + s[..., 8:16] + s[..., 16:24] + s[..., 24:32]) / cnt4
    m4 = jnp.concatenate([m, m, m, m], axis=-1)
    xc = (xf - m4) * bmask
    sq = jnp.sum(xc * xc, axis=(0, 1, 2), keepdims=True)
    v = (sq[..., 0:8] + sq[..., 8:16] + sq[..., 16:24] + sq[..., 24:32]) / cnt4
    g = gu2[...].reshape(1, 1, 1, -1)
    b = bu2[...].reshape(1, 1, 1, -1)
    sc = jax.lax.rsqrt(v + 1e-5) * g
    sh = b - m * sc
    sc4 = jnp.concatenate([sc, sc, sc, sc], axis=-1)
    sh4 = jnp.concatenate([sh, sh, sh, sh], axis=-1)
    o_ref[...] = (xf * sc4 + sh4).astype(jnp.bfloat16)


def kernel(im_x, im_z, cls_gt, ctr_gt,
           d1, d2, d3, d4, d5, d6, d7,
           u7, u6, u5, u4, u3, u2, u1,
           bn_d2__gamma, bn_d2__beta, bn_d3__gamma, bn_d3__beta,
           bn_d4__gamma, bn_d4__beta, bn_d5__gamma, bn_d5__beta,
           bn_d6__gamma, bn_d6__beta, bn_u7__gamma, bn_u7__beta,
           bn_u6__gamma, bn_u6__beta, bn_u5__gamma, bn_u5__beta,
           bn_u4__gamma, bn_u4__beta, bn_u3__gamma, bn_u3__beta,
           bn_u2__gamma, bn_u2__beta):
    x_crop = jax.lax.slice(im_x, (0, 0, 170, 170), (4, 3, 234, 234))
    x_real = (jnp.transpose(x_crop, (0, 2, 3, 1)) / 255.0 - 0.5) / 0.5
    z_crop = jax.lax.slice(im_z, (0, 0, 54, 54), (8, 3, 118, 118))
    z_real = (jnp.transpose(z_crop, (0, 2, 3, 1)) / 255.0 - 0.5) / 0.5

    m_up = jnp.asarray(_interp_mat(64, 128))
    g_in = jnp.einsum('oh,bhwc->bowc', m_up, x_real)
    g_in = jnp.einsum('pw,bowc->bopc', m_up, g_in)
    gz_in = jnp.repeat(jnp.repeat(z_real, 2, axis=1), 2, axis=2)
    gin = jnp.concatenate([g_in.astype(jnp.bfloat16),
                           jnp.zeros((4, 128, 128, 3), jnp.bfloat16),
                           gz_in.astype(jnp.bfloat16)], axis=0)

    xp = jnp.pad(gin, ((0, 0), (1, 1), (1, 1), (0, 0)))
    s2d = xp.reshape(16, 65, 2, 65, 2, 3)
    s2d = jnp.transpose(s2d, (0, 1, 3, 2, 4, 5)).reshape(16, 65, 65, 12)
    acc = None
    for a in range(2):
        for b in range(2):
            v = s2d[:, a:a + 64, b:b + 64, :].reshape(16 * 64 * 64, 12)
            t = jnp.dot(v, d1[a, b], preferred_element_type=jnp.float32)
            acc = t if acc is None else acc + t
    h1 = acc.astype(jnp.bfloat16).reshape(16, 64, 64, 8)

    h1p = jnp.pad(_lrelu(h1), ((0, 0), (1, 1), (1, 1), (0, 0)))
    s2 = h1p.reshape(16, 33, 2, 33, 2, 8)
    s2 = jnp.transpose(s2, (0, 1, 3, 2, 4, 5)).reshape(16, 33, 33, 32)

    weights = [d2, d3, d4, d5, d6, d7, u7, u6, u5, u4, u3, u2]
    bn_params = [bn_d2__gamma, bn_d2__beta, bn_d3__gamma, bn_d3__beta,
                 bn_d4__gamma, bn_d4__beta, bn_d5__gamma, bn_d5__beta,
                 bn_d6__gamma, bn_d6__beta, bn_u7__gamma, bn_u7__beta,
                 bn_u6__gamma, bn_u6__beta, bn_u5__gamma, bn_u5__beta,
                 bn_u4__gamma, bn_u4__beta, bn_u3__gamma, bn_u3__beta,
                 bn_u2__gamma, bn_u2__beta]
    bn_params = [p.reshape(1, -1) for p in bn_params]

    def _full(shape):
        nd = len(shape)
        return pl.BlockSpec(shape, lambda i, _n=nd: (0,) * _n)

    in_specs = ([pl.BlockSpec((8, 33, 33, 32), lambda i: (i, 0, 0, 0))]
                + [_full(w.shape) for w in weights]
                + [_full(p.shape) for p in bn_params])
    f32 = jnp.float32
    scratch_shapes = [
        pltpu.VMEM((8, 34, 34, 16), f32), pltpu.VMEM((8, 18, 18, 32), f32),
        pltpu.VMEM((8, 10, 10, 64), f32), pltpu.VMEM((8, 6, 6, 64), f32),
        pltpu.VMEM((8, 4, 4, 64), f32),
        pltpu.VMEM((8, 2, 2, 64), f32), pltpu.VMEM((8, 4, 4, 64), f32),
        pltpu.VMEM((8, 8, 8, 64), f32), pltpu.VMEM((8, 16, 16, 32), f32),
        pltpu.VMEM((8, 32, 32, 16), f32),
    ]

    u2ph = pl.pallas_call(
        _unet_core_kernel,
        grid=(2,),
        in_specs=in_specs,
        out_specs=pl.BlockSpec((8, 32, 32, 32), lambda i: (i, 0, 0, 0)),
        out_shape=jax.ShapeDtypeStruct((16, 32, 32, 32), jnp.bfloat16),
        scratch_shapes=scratch_shapes,
        compiler_params=pltpu.CompilerParams(
            dimension_semantics=("parallel",)),
    )(s2, *weights, *bn_params)

    u2y = u2ph.reshape(16, 32, 32, 2, 2, 8)
    u2y = jnp.transpose(u2y, (0, 1, 3, 2, 4, 5)).reshape(16, 64, 64, 8)
    u1_in = jnp.concatenate([h1, u2y], axis=-1)
    u1p = jnp.pad(jnp.maximum(u1_in, 0), ((0, 0), (1, 1), (1, 1), (0, 0)))
    acc = None
    for dy in range(3):
        for dx in range(3):
            v = u1p[:, dy:dy + 64, dx:dx + 64, :].reshape(16 * 64 * 64, 16)
            t = jnp.dot(v, u1[dy, dx], preferred_element_type=jnp.float32)
            acc = t if acc is None else acc + t
    acc = jnp.tanh(acc)
    y = acc.astype(jnp.bfloat16).reshape(16, 64, 64, 2, 2, 3)
    gan_out = jnp.transpose(y, (0, 1, 3, 2, 4, 5)).reshape(16, 128, 128, 3)

    m_dn = jnp.asarray(_interp_mat(128, 64))
    g_out = jnp.einsum('oh,bhwc->bowc', m_dn, gan_out[0:4].astype(jnp.float32))
    g_out = jnp.einsum('pw,bowc->bopc', m_dn, g_out)
    gz_out = gan_out[8:16][:, ::2, ::2, :].astype(jnp.float32)

    def composite(g, real):
        x_adv = (g * 0.5 + 0.5) * 0.2
        real01 = real * 0.5 + 0.5
        return jnp.clip(real01 * (1.0 - 0.2) + x_adv, 0.0, 1.0) * 255.0

    x_fake = composite(g_out, x_real)
    z_fake = composite(gz_out, z_real)
    im_x = im_x.at[0:4, :, 170:234, 170:234].set(
        jnp.transpose(x_fake, (0, 3, 1, 2)))
    im_z = im_z.at[0:8, :, 54:118, 54:118].set(
        jnp.transpose(z_fake, (0, 3, 1, 2)))

    yc = np.zeros((4, 17, 17), np.float32)
    yc[:, 9:14, 9:14] = 1.0
    yt = np.zeros((4, 17, 17), np.float32)
    yt[:, 9:14, 9:14] = 0.6
    yt[:, 10:13, 10:13] = 0.7
    yt[:, 11, 11] = 0.9
    cls_gt = cls_gt.at[0:4].set(jnp.asarray(yc.reshape(4, 289, 1)))
    ctr_gt = ctr_gt.at[0:4].set(jnp.asarray(yt.reshape(4, 289, 1)))
    return im_x, im_z, cls_gt, ctr_gt
